# baseline (speedup 1.0000x reference)
import functools

import jax
import jax.numpy as jnp
from jax import lax
from jax.experimental import pallas as pl
from jax.experimental.pallas import tpu as pltpu
from jax.experimental.pallas import tpu_sc as plsc

NB_OBJECTS = 32
DIM_BODY = 10
DIM_OBJECT = 15
D_EDGE = 39
BATCH = 512
N_EDGES = 2048
N_ISO = 4
N_NODES = NB_OBJECTS + N_ISO

NEG = float("-inf")


NW = 32
ROWS_PER_W = BATCH // NW
GCH = 128
NCH = N_EDGES // GCH


def _sc_segmax_body(ef_hbm, perm_hbm, starts_hbm, out_hbm,
                    perm_v, starts_v, idx_v, rows_v, acc_v, sem):
    wid = lax.axis_index("s") * 2 + lax.axis_index("c")
    pltpu.sync_copy(perm_hbm, perm_v)
    pltpu.sync_copy(starts_hbm, starts_v)
    ninf = jnp.full((16,), NEG, jnp.float32)

    HALF = N_EDGES // 2

    rows_per_w = ROWS_PER_W // 2

    def batch_body(i, _):
        base = (wid * rows_per_w + i) * N_EDGES

        st = [starts_v[pl.ds(k * 16, 16)] for k in range(3)]

        def _start(s):
            return st[s // 16][s % 16]

        for h in range(2):
            def idx_body(c, _):
                sl = pl.ds(c * 16, 16)
                idx_v[sl] = (perm_v[pl.ds(h * HALF + c * 16, 16)] + base) * 2
                return 0

            lax.fori_loop(0, HALF // 16, idx_body, 0)

            copies = [
                pltpu.async_copy(
                    ef_hbm.at[idx_v.at[pl.ds(c * GCH, GCH)]],
                    rows_v.at[pl.ds(c * GCH, GCH), :], sem)
                for c in range(HALF // GCH)
            ]
            for cp in copies:
                cp.wait()

            for s in range(NB_OBJECTS):
                lo = jnp.maximum(_start(s), h * HALF) - h * HALF
                hi = jnp.minimum(_start(s + 1), (h + 1) * HALF) - h * HALF

                def red_body(j, carry):
                    a0, a1, a2 = carry
                    return (jnp.maximum(a0, rows_v[j, pl.ds(0, 16)]),
                            jnp.maximum(a1, rows_v[j, pl.ds(16, 16)]),
                            jnp.maximum(a2, rows_v[j, pl.ds(24, 16)]))

                a0, a1, a2 = lax.fori_loop(lo, hi, red_body, (ninf, ninf, ninf))
                if h == 0:
                    acc_v[s, pl.ds(0, 16)] = a0
                    acc_v[s, pl.ds(16, 16)] = a1
                    acc_v[s, pl.ds(24, 16)] = a2
                else:
                    acc_v[s, pl.ds(0, 16)] = jnp.maximum(acc_v[s, pl.ds(0, 16)], a0)
                    acc_v[s, pl.ds(16, 16)] = jnp.maximum(acc_v[s, pl.ds(16, 16)], a1)
                    acc_v[s, pl.ds(24, 16)] = jnp.maximum(acc_v[s, pl.ds(24, 16)], a2)

        pltpu.sync_copy(acc_v, out_hbm.at[wid * rows_per_w + i])
        return 0

    lax.fori_loop(0, rows_per_w, batch_body, 0)


def _sorted_positions(edges_to):
    onehot = (edges_to[:, None] == jnp.arange(NB_OBJECTS, dtype=jnp.int32)[None, :])
    oh = onehot.astype(jnp.int32)
    pref = jnp.cumsum(oh, axis=0)
    counts = pref[-1]
    starts0 = jnp.concatenate(
        [jnp.zeros((1,), jnp.int32), jnp.cumsum(counts, dtype=jnp.int32)])
    rank = jnp.sum(oh * pref, axis=1) - 1
    seg_start = jnp.sum(oh * starts0[None, :NB_OBJECTS], axis=1)
    pos = (rank + seg_start).astype(jnp.int32)
    perm = jnp.zeros((N_EDGES,), jnp.int32).at[pos].set(
        jnp.arange(N_EDGES, dtype=jnp.int32))
    starts = jnp.concatenate([starts0, jnp.zeros((15,), jnp.int32)])
    return perm, starts


def _segment_max(edge_features, edges_to):
    perm, starts = _sorted_positions(edges_to)

    mesh = plsc.VectorSubcoreMesh(core_axis_name="c", subcore_axis_name="s")
    run = functools.partial(
        pl.kernel,
        mesh=mesh,
        compiler_params=pltpu.CompilerParams(use_tc_tiling_on_sc=False),
        out_type=jax.ShapeDtypeStruct((BATCH // 2, NB_OBJECTS, 40), jnp.float32),
        scratch_types=[
            pltpu.VMEM((N_EDGES,), jnp.int32),
            pltpu.VMEM((48,), jnp.int32),
            pltpu.VMEM((N_EDGES // 2,), jnp.int32),
            pltpu.VMEM((N_EDGES // 2, 64), jnp.float32),
            pltpu.VMEM((NB_OBJECTS, 40), jnp.float32),
            pltpu.SemaphoreType.DMA,
        ],
    )(_sc_segmax_body)
    halves = []
    for hb in range(2):
        ef_h = edge_features[hb * (BATCH // 2):(hb + 1) * (BATCH // 2)]
        ef2d = jnp.pad(ef_h, ((0, 0), (0, 0), (0, 89))).reshape(
            BATCH // 2 * N_EDGES * 2, 64)
        halves.append(run(ef2d, perm, starts))
    return jnp.concatenate(halves, axis=0)


MB = 128
MB_STEPS = BATCH // MB


def _mlp_body(obs_ref, incb_ref, iso_ref, isof_ref, w1b_ref, w1o_ref, w1i_ref,
              b1_ref, w2_ref, b2_ref, rw1_ref, rb1_ref, rw2_ref, rb2_ref,
              mw_ref, mb_ref, lw_ref, lb_ref, mean_ref, log_ref):
    def dot(a, b):
        return jnp.dot(a, b, preferred_element_type=jnp.float32)

    body_part = dot(obs_ref[:, pl.ds(0, DIM_BODY)], w1b_ref[...]) + b1_ref[...]
    pooled = None
    for n in range(N_NODES):
        if n < NB_OBJECTS:
            x_o = obs_ref[:, pl.ds(DIM_BODY + n * DIM_OBJECT, DIM_OBJECT)]
            x_i = incb_ref[:, n, pl.ds(0, D_EDGE)]
        else:
            x_o = iso_ref[:, n - NB_OBJECTS, :]
            x_i = isof_ref[:, n - NB_OBJECTS, :]
        h = jnp.maximum(body_part + dot(x_o, w1o_ref[...]) + dot(x_i, w1i_ref[...]),
                        0.0)
        h2 = jnp.maximum(dot(h, w2_ref[...]) + b2_ref[...], 0.0)
        pooled = h2 if pooled is None else jnp.maximum(pooled, h2)

    r = jnp.maximum(dot(pooled, rw1_ref[...]) + rb1_ref[...], 0.0)
    r = jnp.maximum(dot(r, rw2_ref[...]) + rb2_ref[...], 0.0)
    mean_ref[...] = dot(r, mw_ref[...]) + mb_ref[...]
    log_ref[...] = jnp.clip(dot(r, lw_ref[...]) + lb_ref[...], -20.0, 2.0)


def _full(shape):
    return pl.BlockSpec(shape, lambda b: tuple(0 for _ in shape))


def _bblk(shape):
    return pl.BlockSpec(shape, lambda b: (b,) + tuple(0 for _ in shape[1:]))


def _mlp(obs, incb, iso, isof, w1b, w1o, w1i, b1, w2, b2,
         rw1, rb1, rw2, rb2, mw, mb, lw, lb):
    return pl.pallas_call(
        _mlp_body,
        grid=(MB_STEPS,),
        in_specs=[
            _bblk((MB, DIM_BODY + NB_OBJECTS * DIM_OBJECT)),
            _bblk((MB, NB_OBJECTS, 40)),
            _bblk((MB, N_ISO, DIM_OBJECT)),
            _bblk((MB, N_ISO, D_EDGE)),
            _full(w1b.shape), _full(w1o.shape), _full(w1i.shape),
            _full(b1.shape), _full(w2.shape), _full(b2.shape),
            _full(rw1.shape), _full(rb1.shape), _full(rw2.shape), _full(rb2.shape),
            _full(mw.shape), _full(mb.shape), _full(lw.shape), _full(lb.shape),
        ],
        out_specs=[_bblk((MB, 8)), _bblk((MB, 8))],
        out_shape=[jax.ShapeDtypeStruct((BATCH, 8), jnp.float32),
                   jax.ShapeDtypeStruct((BATCH, 8), jnp.float32)],
    )(obs, incb, iso, isof, w1b, w1o, w1i, b1, w2, b2,
      rw1, rb1, rw2, rb2, mw, mb, lw, lb)


def kernel(obs, edge_features, edges_to, isolated_nodes, isolated_nodes_features,
           phi_w1, phi_b1, phi_w2, phi_b2, rho_w1, rho_b1, rho_w2, rho_b2,
           mean_w, mean_b, log_w, log_b):
    incoming = _segment_max(edge_features, edges_to)

    w1b = phi_w1[:DIM_BODY]
    w1o = phi_w1[DIM_BODY:DIM_BODY + DIM_OBJECT]
    w1i = phi_w1[DIM_BODY + DIM_OBJECT:]

    mean, log_std = _mlp(
        obs, incoming, isolated_nodes, isolated_nodes_features,
        w1b, w1o, w1i, phi_b1.reshape(1, -1),
        phi_w2, phi_b2.reshape(1, -1), rho_w1, rho_b1.reshape(1, -1),
        rho_w2, rho_b2.reshape(1, -1), mean_w, mean_b.reshape(1, -1),
        log_w, log_b.reshape(1, -1))
    return (mean, log_std)


# speedup vs baseline: 1.1403x; 1.1403x over previous
import functools

import jax
import jax.numpy as jnp
from jax import lax
from jax.experimental import pallas as pl
from jax.experimental.pallas import tpu as pltpu
from jax.experimental.pallas import tpu_sc as plsc

NB_OBJECTS = 32
DIM_BODY = 10
DIM_OBJECT = 15
D_EDGE = 39
BATCH = 512
N_EDGES = 2048
N_ISO = 4
N_NODES = NB_OBJECTS + N_ISO

NEG = float("-inf")


NW = 32
ROWS_PER_W = BATCH // NW
GCH = 128
NCH = N_EDGES // GCH


def _sc_segmax_body(ef_hbm, perm_hbm, starts_hbm, out_hbm,
                    perm_v, starts_v, idx_v, rows_v, acc_v, sem):
    wid = lax.axis_index("s") * 2 + lax.axis_index("c")
    pltpu.sync_copy(perm_hbm, perm_v)
    pltpu.sync_copy(starts_hbm, starts_v)
    ninf = jnp.full((16,), NEG, jnp.float32)

    HALF = N_EDGES // 2

    def batch_body(i, _):
        base = (wid * ROWS_PER_W + i) * N_EDGES

        st = [starts_v[pl.ds(k * 16, 16)] for k in range(3)]

        def _start(s):
            return st[s // 16][s % 16]

        for h in range(2):
            def idx_body(c, _):
                sl = pl.ds(c * 16, 16)
                idx_v[sl] = (perm_v[pl.ds(h * HALF + c * 16, 16)] + base) * 2
                return 0

            lax.fori_loop(0, HALF // 16, idx_body, 0)

            copies = [
                pltpu.async_copy(
                    ef_hbm.at[idx_v.at[pl.ds(c * GCH, GCH)]],
                    rows_v.at[pl.ds(c * GCH, GCH), :], sem)
                for c in range(HALF // GCH)
            ]
            for cp in copies:
                cp.wait()

            for s in range(NB_OBJECTS):
                lo = jnp.maximum(_start(s), h * HALF) - h * HALF
                hi = jnp.minimum(_start(s + 1), (h + 1) * HALF) - h * HALF

                def red_body(j, carry):
                    a0, a1, a2 = carry
                    return (jnp.maximum(a0, rows_v[j, pl.ds(0, 16)]),
                            jnp.maximum(a1, rows_v[j, pl.ds(16, 16)]),
                            jnp.maximum(a2, rows_v[j, pl.ds(24, 16)]))

                a0, a1, a2 = lax.fori_loop(lo, hi, red_body, (ninf, ninf, ninf))
                if h == 0:
                    acc_v[s, pl.ds(0, 16)] = a0
                    acc_v[s, pl.ds(16, 16)] = a1
                    acc_v[s, pl.ds(24, 16)] = a2
                else:
                    acc_v[s, pl.ds(0, 16)] = jnp.maximum(acc_v[s, pl.ds(0, 16)], a0)
                    acc_v[s, pl.ds(16, 16)] = jnp.maximum(acc_v[s, pl.ds(16, 16)], a1)
                    acc_v[s, pl.ds(24, 16)] = jnp.maximum(acc_v[s, pl.ds(24, 16)], a2)

        pltpu.sync_copy(acc_v, out_hbm.at[wid * ROWS_PER_W + i])
        return 0

    lax.fori_loop(0, ROWS_PER_W, batch_body, 0)


def _sorted_positions(edges_to):
    onehot = (edges_to[:, None] == jnp.arange(NB_OBJECTS, dtype=jnp.int32)[None, :])
    oh = onehot.astype(jnp.int32)
    pref = jnp.cumsum(oh, axis=0)
    counts = pref[-1]
    starts0 = jnp.concatenate(
        [jnp.zeros((1,), jnp.int32), jnp.cumsum(counts, dtype=jnp.int32)])
    rank = jnp.sum(oh * pref, axis=1) - 1
    seg_start = jnp.sum(oh * starts0[None, :NB_OBJECTS], axis=1)
    pos = (rank + seg_start).astype(jnp.int32)
    perm = jnp.zeros((N_EDGES,), jnp.int32).at[pos].set(
        jnp.arange(N_EDGES, dtype=jnp.int32))
    starts = jnp.concatenate([starts0, jnp.zeros((15,), jnp.int32)])
    return perm, starts


def _segment_max(edge_features, edges_to):
    perm, starts = _sorted_positions(edges_to)
    ef2d = jnp.pad(edge_features, ((0, 0), (0, 0), (0, 89))).reshape(
        BATCH * N_EDGES * 2, 64)

    mesh = plsc.VectorSubcoreMesh(core_axis_name="c", subcore_axis_name="s")
    run = functools.partial(
        pl.kernel,
        mesh=mesh,
        compiler_params=pltpu.CompilerParams(use_tc_tiling_on_sc=False),
        out_type=jax.ShapeDtypeStruct((BATCH, NB_OBJECTS, 40), jnp.float32),
        scratch_types=[
            pltpu.VMEM((N_EDGES,), jnp.int32),
            pltpu.VMEM((48,), jnp.int32),
            pltpu.VMEM((N_EDGES // 2,), jnp.int32),
            pltpu.VMEM((N_EDGES // 2, 64), jnp.float32),
            pltpu.VMEM((NB_OBJECTS, 40), jnp.float32),
            pltpu.SemaphoreType.DMA,
        ],
    )(_sc_segmax_body)
    return run(ef2d, perm, starts)


MB = 128
MB_STEPS = BATCH // MB


def _mlp_body(obs_ref, incb_ref, iso_ref, isof_ref, w1b_ref, w1o_ref, w1i_ref,
              b1_ref, w2_ref, b2_ref, rw1_ref, rb1_ref, rw2_ref, rb2_ref,
              mw_ref, mb_ref, lw_ref, lb_ref, mean_ref, log_ref):
    def dot(a, b):
        return jnp.dot(a, b, preferred_element_type=jnp.float32)

    body_part = dot(obs_ref[:, pl.ds(0, DIM_BODY)], w1b_ref[...]) + b1_ref[...]
    pooled = None
    for n in range(N_NODES):
        if n < NB_OBJECTS:
            x_o = obs_ref[:, pl.ds(DIM_BODY + n * DIM_OBJECT, DIM_OBJECT)]
            x_i = incb_ref[:, n, pl.ds(0, D_EDGE)]
        else:
            x_o = iso_ref[:, n - NB_OBJECTS, :]
            x_i = isof_ref[:, n - NB_OBJECTS, :]
        h = jnp.maximum(body_part + dot(x_o, w1o_ref[...]) + dot(x_i, w1i_ref[...]),
                        0.0)
        h2 = jnp.maximum(dot(h, w2_ref[...]) + b2_ref[...], 0.0)
        pooled = h2 if pooled is None else jnp.maximum(pooled, h2)

    r = jnp.maximum(dot(pooled, rw1_ref[...]) + rb1_ref[...], 0.0)
    r = jnp.maximum(dot(r, rw2_ref[...]) + rb2_ref[...], 0.0)
    mean_ref[...] = dot(r, mw_ref[...]) + mb_ref[...]
    log_ref[...] = jnp.clip(dot(r, lw_ref[...]) + lb_ref[...], -20.0, 2.0)


def _full(shape):
    return pl.BlockSpec(shape, lambda b: tuple(0 for _ in shape))


def _bblk(shape):
    return pl.BlockSpec(shape, lambda b: (b,) + tuple(0 for _ in shape[1:]))


def _mlp(obs, incb, iso, isof, w1b, w1o, w1i, b1, w2, b2,
         rw1, rb1, rw2, rb2, mw, mb, lw, lb):
    return pl.pallas_call(
        _mlp_body,
        grid=(MB_STEPS,),
        in_specs=[
            _bblk((MB, DIM_BODY + NB_OBJECTS * DIM_OBJECT)),
            _bblk((MB, NB_OBJECTS, 40)),
            _bblk((MB, N_ISO, DIM_OBJECT)),
            _bblk((MB, N_ISO, D_EDGE)),
            _full(w1b.shape), _full(w1o.shape), _full(w1i.shape),
            _full(b1.shape), _full(w2.shape), _full(b2.shape),
            _full(rw1.shape), _full(rb1.shape), _full(rw2.shape), _full(rb2.shape),
            _full(mw.shape), _full(mb.shape), _full(lw.shape), _full(lb.shape),
        ],
        out_specs=[_bblk((MB, 8)), _bblk((MB, 8))],
        out_shape=[jax.ShapeDtypeStruct((BATCH, 8), jnp.float32),
                   jax.ShapeDtypeStruct((BATCH, 8), jnp.float32)],
    )(obs, incb, iso, isof, w1b, w1o, w1i, b1, w2, b2,
      rw1, rb1, rw2, rb2, mw, mb, lw, lb)


def kernel(obs, edge_features, edges_to, isolated_nodes, isolated_nodes_features,
           phi_w1, phi_b1, phi_w2, phi_b2, rho_w1, rho_b1, rho_w2, rho_b2,
           mean_w, mean_b, log_w, log_b):
    incoming = _segment_max(edge_features, edges_to)

    w1b = phi_w1[:DIM_BODY]
    w1o = phi_w1[DIM_BODY:DIM_BODY + DIM_OBJECT]
    w1i = phi_w1[DIM_BODY + DIM_OBJECT:]

    mean, log_std = _mlp(
        obs, incoming, isolated_nodes, isolated_nodes_features,
        w1b, w1o, w1i, phi_b1.reshape(1, -1),
        phi_w2, phi_b2.reshape(1, -1), rho_w1, rho_b1.reshape(1, -1),
        rho_w2, rho_b2.reshape(1, -1), mean_w, mean_b.reshape(1, -1),
        log_w, log_b.reshape(1, -1))
    return (mean, log_std)
